# Initial kernel scaffold; baseline (speedup 1.0000x reference)
import jax
import jax.numpy as jnp
from jax.experimental import pallas as pl


def kernel(rpn_cls_logits, rpn_bbox_reg, anchor_labels, anchor_gt_boxes):
    raise NotImplementedError("write your pallas kernel here")


# speedup vs baseline: 1.1343x; 1.1343x over previous
import jax
import jax.numpy as jnp
from jax.experimental import pallas as pl
from jax.experimental.pallas import tpu as pltpu

_CLS_W = 1.0
_BBOX_W = 10.0
_BS = 8
_A = 9
_HW = 2500
_N = _A * _HW


def _loss_body(logits_ref, labels_a_ref, bbox_ref, gt_ref, labels_p_ref,
               out_ref):
    lg = logits_ref[...]
    tg = labels_a_ref[...]
    bce_sum = jnp.sum(
        jnp.maximum(lg, 0.0) - lg * tg + jnp.log1p(jnp.exp(-jnp.abs(lg))))

    mp = labels_p_ref[...]
    npos = jnp.sum(mp)

    a_i = jax.lax.broadcasted_iota(jnp.int32, (_A, 4 * _A), 0)
    ch_i = jax.lax.broadcasted_iota(jnp.int32, (_A, 4 * _A), 1)
    pmat = (a_i == ch_i // 4).astype(jnp.float32)
    mask36 = jax.lax.dot_general(
        mp, pmat, dimension_numbers=(((2,), (0,)), ((), ())),
        preferred_element_type=jnp.float32)

    bt = jnp.transpose(bbox_ref[...], (0, 2, 1))
    diff = bt - gt_ref[...]
    ad = jnp.abs(diff)
    sl1 = jnp.where(ad < 1.0, 0.5 * diff * diff, ad - 0.5)
    masked_sum = jnp.sum(sl1 * mask36)

    cls_loss = bce_sum / (_BS * _N)
    denom = jnp.maximum(2.0 * npos, 1.0)
    bbox_loss = jnp.where(npos > 0.0, masked_sum / denom, 0.0)
    out_ref[0, 0] = _CLS_W * cls_loss + _BBOX_W * bbox_loss


def kernel(rpn_cls_logits, rpn_bbox_reg, anchor_labels, anchor_gt_boxes):
    labels_f = anchor_labels.astype(jnp.float32)
    logits = rpn_cls_logits.reshape(_BS * _A, _HW)
    labels_a = labels_f.reshape(_BS * _A, _HW)
    labels_p = labels_f.reshape(_BS, _HW, _A)
    bbox = rpn_bbox_reg.reshape(_BS, 4 * _A, _HW)
    gt = anchor_gt_boxes.reshape(_BS, _HW, 4 * _A)

    out = pl.pallas_call(
        _loss_body,
        out_shape=jax.ShapeDtypeStruct((1, 1), jnp.float32),
        out_specs=pl.BlockSpec(memory_space=pltpu.SMEM),
    )(logits, labels_a, bbox, gt, labels_p)
    return out[0, 0]
